# baseline (speedup 1.0000x reference)
import functools

import jax
import jax.numpy as jnp
from jax import lax
from jax.experimental import pallas as pl
from jax.experimental.pallas import tpu as pltpu
from jax.experimental.pallas import tpu_sc as plsc

B = 16384
D = 16
H = 8
NW = 32
BW = B // NW
NG = BW // 16

P_W1 = 0
P_B1 = 384
P_W2 = 392
P_B2 = 400
P_LEN = 408


@functools.partial(
    pl.kernel,
    mesh=plsc.VectorSubcoreMesh(core_axis_name="c", subcore_axis_name="s"),
    out_type=jax.ShapeDtypeStruct((B,), jnp.float32),
    scratch_types=[
        pltpu.VMEM((BW,), jnp.int32),
        pltpu.VMEM((BW,), jnp.int32),
        pltpu.VMEM((BW, D), jnp.float32),
        pltpu.VMEM((BW, D), jnp.float32),
        pltpu.VMEM((P_LEN, 16), jnp.float32),
        pltpu.VMEM((BW,), jnp.float32),
        pltpu.SemaphoreType.DMA,
        pltpu.SemaphoreType.DMA,
    ],
    compiler_params=pltpu.CompilerParams(
        needs_layout_passes=False, use_tc_tiling_on_sc=False),
)
def _hhgr_sc(uin_hbm, iin_hbm, utab_hbm, itab_hbm, params_hbm, out_hbm,
             uidx_v, iidx_v, urows_v, irows_v, pv, out_v, sem_u, sem_i):
    wid = lax.axis_index("s") * 2 + lax.axis_index("c")
    base = wid * BW

    pltpu.sync_copy(uin_hbm.at[pl.ds(base, BW)], uidx_v)
    pltpu.sync_copy(iin_hbm.at[pl.ds(base, BW)], iidx_v)
    pltpu.sync_copy(params_hbm, pv)

    cu = pltpu.async_copy(utab_hbm.at[uidx_v], urows_v, sem_u)
    ci = pltpu.async_copy(itab_hbm.at[iidx_v], irows_v, sem_i)
    cu.wait()
    ci.wait()

    iota = lax.iota(jnp.int32, 16)

    def wbcast(idx):
        return pv[idx]

    b1v = [wbcast(P_B1 + j) for j in range(H)]
    w2v = [wbcast(P_W2 + j) for j in range(H)]
    b2v = wbcast(P_B2)

    def group(g, carry):
        row = g * 16 + iota
        h = list(b1v)
        for d in range(D):
            col = jnp.full((16,), d, jnp.int32)
            u_d = plsc.load_gather(urows_v, [row, col])
            i_d = plsc.load_gather(irows_v, [row, col])
            e_d = u_d * i_d
            for j in range(H):
                h[j] = (h[j]
                        + e_d * wbcast(P_W1 + d * H + j)
                        + u_d * wbcast(P_W1 + (D + d) * H + j)
                        + i_d * wbcast(P_W1 + (2 * D + d) * H + j))
        logit = b2v
        for j in range(H):
            logit = logit + jnp.maximum(h[j], 0.0) * w2v[j]
        out_v[pl.ds(g * 16, 16)] = 1.0 / (1.0 + jnp.exp(-logit))
        return carry

    lax.fori_loop(0, NG, group, 0)

    pltpu.sync_copy(out_v, out_hbm.at[pl.ds(base, BW)])


def kernel(user_inputs, item_inputs, user_table, item_table, W1, b1, W2, b2):
    params = jnp.concatenate([
        W1.reshape(-1),
        b1.reshape(-1),
        W2.reshape(-1),
        b2.reshape(-1),
        jnp.zeros((P_LEN - 401,), jnp.float32),
    ])
    params = jnp.broadcast_to(params[:, None], (P_LEN, 16))
    y = _hhgr_sc(user_inputs.astype(jnp.int32), item_inputs.astype(jnp.int32),
                 user_table, item_table, params)
    return y.reshape(B, 1)


# speedup vs baseline: 3.4953x; 3.4953x over previous
import functools

import jax
import jax.numpy as jnp
from jax import lax
from jax.experimental import pallas as pl
from jax.experimental.pallas import tpu as pltpu
from jax.experimental.pallas import tpu_sc as plsc

B = 16384
D = 16
H = 8
NW = 32
BW = B // NW
NG = BW // 16

P_B1 = 384
P_W2 = 392
P_B2 = 400


@functools.partial(
    pl.kernel,
    mesh=plsc.VectorSubcoreMesh(core_axis_name="c", subcore_axis_name="s"),
    out_type=jax.ShapeDtypeStruct((B,), jnp.float32),
    scratch_types=[
        pltpu.VMEM((BW,), jnp.int32),
        pltpu.VMEM((BW,), jnp.int32),
        pltpu.VMEM((D, 8 * 128), jnp.float32),
        pltpu.VMEM((D, 16), jnp.float32),
        pltpu.VMEM((D, 16), jnp.float32),
        pltpu.VMEM((4, 128), jnp.float32),
        pltpu.VMEM((BW,), jnp.float32),
        pltpu.SemaphoreType.DMA,
    ],
    compiler_params=pltpu.CompilerParams(needs_layout_passes=False),
)
def _hhgr_sc(uin_hbm, iin_hbm, utabt_hbm, itabt_hbm, params_hbm, out_hbm,
             uidx_v, iidx_v, blk_v, ustg_v, istg_v, pv, out_v, sem):
    wid = lax.axis_index("s") * 2 + lax.axis_index("c")
    base = wid * BW

    pltpu.sync_copy(uin_hbm.at[pl.ds(base, BW)], uidx_v)
    pltpu.sync_copy(iin_hbm.at[pl.ds(base, BW)], iidx_v)
    pltpu.sync_copy(params_hbm, pv)

    iota = lax.iota(jnp.int32, 16)

    def wave(tab_hbm, idxvec, lanevec, stg, half):
        for e in range(8):
            blkbase = pl.multiple_of(
                idxvec[half * 8 + e] & ~jnp.int32(127), 128)
            pltpu.async_copy(
                tab_hbm.at[:, pl.ds(blkbase, 128)],
                blk_v.at[:, pl.ds(e * 128, 128)], sem)
        pltpu.make_async_copy(tab_hbm.at[:, pl.ds(0, 8 * 128)],
                              blk_v, sem).wait()
        for e in range(8):
            col = jnp.full((16,), e * 128, jnp.int32) + lanevec[half * 8 + e]
            vals = plsc.load_gather(blk_v, [iota, col])
            plsc.store_scatter(
                stg, [iota, jnp.full((16,), half * 8 + e, jnp.int32)], vals)

    def wb(w):
        return plsc.load_gather(
            pv, [jnp.full((16,), w // 128, jnp.int32),
                 jnp.full((16,), w % 128, jnp.int32)])

    def rnd(g, carry):
        uv = uidx_v[pl.ds(g * 16, 16)]
        iv = iidx_v[pl.ds(g * 16, 16)]
        ulane = uv & 127
        ilane = iv & 127
        wave(utabt_hbm, uv, ulane, ustg_v, 0)
        wave(utabt_hbm, uv, ulane, ustg_v, 1)
        wave(itabt_hbm, iv, ilane, istg_v, 0)
        wave(itabt_hbm, iv, ilane, istg_v, 1)

        h = [wb(P_B1 + j) for j in range(H)]
        for d in range(D):
            u_d = ustg_v[d]
            i_d = istg_v[d]
            e_d = u_d * i_d
            for j in range(H):
                h[j] = (h[j]
                        + e_d * wb(d * H + j)
                        + u_d * wb((D + d) * H + j)
                        + i_d * wb((2 * D + d) * H + j))
        logit = wb(P_B2)
        for j in range(H):
            logit = logit + jnp.maximum(h[j], 0.0) * wb(P_W2 + j)
        out_v[pl.ds(g * 16, 16)] = 1.0 / (1.0 + jnp.exp(-logit))
        return carry

    lax.fori_loop(0, NG, rnd, 0)

    pltpu.sync_copy(out_v, out_hbm.at[pl.ds(base, BW)])


def kernel(user_inputs, item_inputs, user_table, item_table, W1, b1, W2, b2):
    flat = jnp.concatenate([
        W1.reshape(-1),
        b1.reshape(-1),
        W2.reshape(-1),
        b2.reshape(-1),
        jnp.zeros((111,), jnp.float32),
    ])
    params = flat.reshape(4, 128)
    y = _hhgr_sc(user_inputs.astype(jnp.int32), item_inputs.astype(jnp.int32),
                 user_table.T, item_table.T, params)
    return y.reshape(B, 1)


# speedup vs baseline: 4.7753x; 1.3662x over previous
import functools

import jax
import jax.numpy as jnp
from jax import lax
from jax.experimental import pallas as pl
from jax.experimental.pallas import tpu as pltpu
from jax.experimental.pallas import tpu_sc as plsc

B = 16384
D = 16
H = 8
NW = 32
BW = B // NW
NG = BW // 16

P_B1 = 384
P_W2 = 392
P_B2 = 400


@functools.partial(
    pl.kernel,
    mesh=plsc.VectorSubcoreMesh(core_axis_name="c", subcore_axis_name="s"),
    out_type=jax.ShapeDtypeStruct((B,), jnp.float32),
    scratch_types=[
        pltpu.VMEM((BW,), jnp.int32),
        pltpu.VMEM((BW,), jnp.int32),
        pltpu.VMEM((D, 8 * 128), jnp.float32),
        pltpu.VMEM((D, 8 * 128), jnp.float32),
        pltpu.VMEM((D, 16), jnp.float32),
        pltpu.VMEM((D, 16), jnp.float32),
        pltpu.VMEM((4, 128), jnp.float32),
        pltpu.VMEM((BW,), jnp.float32),
        pltpu.SemaphoreType.DMA,
        pltpu.SemaphoreType.DMA,
    ],
    compiler_params=pltpu.CompilerParams(needs_layout_passes=False),
)
def _hhgr_sc(uin_hbm, iin_hbm, utabt_hbm, itabt_hbm, params_hbm, out_hbm,
             uidx_v, iidx_v, blka_v, blkb_v, ustg_v, istg_v, pv, out_v,
             sem_a, sem_b):
    wid = lax.axis_index("s") * 2 + lax.axis_index("c")
    base = wid * BW

    pltpu.sync_copy(uin_hbm.at[pl.ds(base, BW)], uidx_v)
    pltpu.sync_copy(iin_hbm.at[pl.ds(base, BW)], iidx_v)
    pltpu.sync_copy(params_hbm, pv)

    iota = lax.iota(jnp.int32, 16)

    def fire(tab_hbm, idxvec, half, buf, sem):
        for e in range(8):
            blkbase = pl.multiple_of(
                idxvec[half * 8 + e] & ~jnp.int32(127), 128)
            pltpu.async_copy(
                tab_hbm.at[:, pl.ds(blkbase, 128)],
                buf.at[:, pl.ds(e * 128, 128)], sem)

    def drain(buf, sem):
        pltpu.make_async_copy(utabt_hbm.at[:, pl.ds(0, 8 * 128)],
                              buf, sem).wait()

    def ext(lanevec, half, buf, stg):
        for e in range(8):
            col = jnp.full((16,), e * 128, jnp.int32) + lanevec[half * 8 + e]
            vals = plsc.load_gather(buf, [iota, col])
            plsc.store_scatter(
                stg, [iota, jnp.full((16,), half * 8 + e, jnp.int32)], vals)

    def wb(w):
        return plsc.load_gather(
            pv, [jnp.full((16,), w // 128, jnp.int32),
                 jnp.full((16,), w % 128, jnp.int32)])

    uv0 = uidx_v[pl.ds(0, 16)]
    fire(utabt_hbm, uv0, 0, blka_v, sem_a)

    def rnd(g, carry):
        uv = uidx_v[pl.ds(g * 16, 16)]
        iv = iidx_v[pl.ds(g * 16, 16)]
        ulane = uv & 127
        ilane = iv & 127

        fire(utabt_hbm, uv, 1, blkb_v, sem_b)
        drain(blka_v, sem_a)
        ext(ulane, 0, blka_v, ustg_v)

        fire(itabt_hbm, iv, 0, blka_v, sem_a)
        drain(blkb_v, sem_b)
        ext(ulane, 1, blkb_v, ustg_v)

        fire(itabt_hbm, iv, 1, blkb_v, sem_b)
        drain(blka_v, sem_a)
        ext(ilane, 0, blka_v, istg_v)

        gn = jnp.minimum(g + 1, NG - 1)
        uvn = uidx_v[pl.ds(gn * 16, 16)]
        fire(utabt_hbm, uvn, 0, blka_v, sem_a)

        drain(blkb_v, sem_b)
        ext(ilane, 1, blkb_v, istg_v)

        h = [wb(P_B1 + j) for j in range(H)]
        for d in range(D):
            u_d = ustg_v[d]
            i_d = istg_v[d]
            e_d = u_d * i_d
            for j in range(H):
                h[j] = (h[j]
                        + e_d * wb(d * H + j)
                        + u_d * wb((D + d) * H + j)
                        + i_d * wb((2 * D + d) * H + j))
        logit = wb(P_B2)
        for j in range(H):
            logit = logit + jnp.maximum(h[j], 0.0) * wb(P_W2 + j)
        out_v[pl.ds(g * 16, 16)] = 1.0 / (1.0 + jnp.exp(-logit))
        return carry

    lax.fori_loop(0, NG, rnd, 0)

    drain(blka_v, sem_a)
    pltpu.sync_copy(out_v, out_hbm.at[pl.ds(base, BW)])


def kernel(user_inputs, item_inputs, user_table, item_table, W1, b1, W2, b2):
    flat = jnp.concatenate([
        W1.reshape(-1),
        b1.reshape(-1),
        W2.reshape(-1),
        b2.reshape(-1),
        jnp.zeros((111,), jnp.float32),
    ])
    params = flat.reshape(4, 128)
    y = _hhgr_sc(user_inputs.astype(jnp.int32), item_inputs.astype(jnp.int32),
                 user_table.T, item_table.T, params)
    return y.reshape(B, 1)


# speedup vs baseline: 6.4138x; 1.3431x over previous
import functools

import jax
import jax.numpy as jnp
from jax import lax
from jax.experimental import pallas as pl
from jax.experimental.pallas import tpu as pltpu
from jax.experimental.pallas import tpu_sc as plsc

B = 16384
D = 16
H = 8
NW = 32
BW = B // NW
NG = BW // 16

P_B1 = 384
P_W2 = 392
P_B2 = 400


@functools.partial(
    pl.kernel,
    mesh=plsc.VectorSubcoreMesh(core_axis_name="c", subcore_axis_name="s"),
    out_type=jax.ShapeDtypeStruct((B,), jnp.float32),
    scratch_types=[
        pltpu.VMEM((BW,), jnp.int32),
        pltpu.VMEM((BW,), jnp.int32),
        pltpu.VMEM((D, 8 * 128), jnp.float32),
        pltpu.VMEM((D, 8 * 128), jnp.float32),
        pltpu.VMEM((D, 8 * 128), jnp.float32),
        pltpu.VMEM((D, 8 * 128), jnp.float32),
        pltpu.VMEM((D, 16), jnp.float32),
        pltpu.VMEM((D, 16), jnp.float32),
        pltpu.VMEM((4, 128), jnp.float32),
        pltpu.VMEM((BW,), jnp.float32),
        pltpu.SemaphoreType.DMA,
        pltpu.SemaphoreType.DMA,
        pltpu.SemaphoreType.DMA,
        pltpu.SemaphoreType.DMA,
    ],
    compiler_params=pltpu.CompilerParams(needs_layout_passes=False),
)
def _hhgr_sc(uin_hbm, iin_hbm, utabt_hbm, itabt_hbm, params_hbm, out_hbm,
             uidx_v, iidx_v, blk0_v, blk1_v, blk2_v, blk3_v,
             ustg_v, istg_v, pv, out_v, sem0, sem1, sem2, sem3):
    wid = lax.axis_index("s") * 2 + lax.axis_index("c")
    base = wid * BW

    pltpu.sync_copy(uin_hbm.at[pl.ds(base, BW)], uidx_v)
    pltpu.sync_copy(iin_hbm.at[pl.ds(base, BW)], iidx_v)
    pltpu.sync_copy(params_hbm, pv)

    iota = lax.iota(jnp.int32, 16)
    bufs = [blk0_v, blk1_v, blk2_v, blk3_v]
    sems = [sem0, sem1, sem2, sem3]

    def fire(tab_hbm, idxvec, half, w):
        for e in range(8):
            blkbase = pl.multiple_of(
                idxvec[half * 8 + e] & ~jnp.int32(127), 128)
            pltpu.async_copy(
                tab_hbm.at[:, pl.ds(blkbase, 128)],
                bufs[w].at[:, pl.ds(e * 128, 128)], sems[w])

    def drain(w):
        pltpu.make_async_copy(utabt_hbm.at[:, pl.ds(0, 8 * 128)],
                              bufs[w], sems[w]).wait()

    def ext(lanevec, half, w, stg):
        for e in range(8):
            col = jnp.full((16,), e * 128, jnp.int32) + lanevec[half * 8 + e]
            vals = plsc.load_gather(bufs[w], [iota, col])
            plsc.store_scatter(
                stg, [iota, jnp.full((16,), half * 8 + e, jnp.int32)], vals)

    def wb(w):
        return plsc.load_gather(
            pv, [jnp.full((16,), w // 128, jnp.int32),
                 jnp.full((16,), w % 128, jnp.int32)])

    def fire_round(uv, iv):
        fire(utabt_hbm, uv, 0, 0)
        fire(utabt_hbm, uv, 1, 1)
        fire(itabt_hbm, iv, 0, 2)
        fire(itabt_hbm, iv, 1, 3)

    fire_round(uidx_v[pl.ds(0, 16)], iidx_v[pl.ds(0, 16)])

    def rnd(g, carry):
        uv = uidx_v[pl.ds(g * 16, 16)]
        iv = iidx_v[pl.ds(g * 16, 16)]
        ulane = uv & 127
        ilane = iv & 127
        gn = jnp.minimum(g + 1, NG - 1)
        uvn = uidx_v[pl.ds(gn * 16, 16)]
        ivn = iidx_v[pl.ds(gn * 16, 16)]

        drain(0)
        ext(ulane, 0, 0, ustg_v)
        fire(utabt_hbm, uvn, 0, 0)
        drain(1)
        ext(ulane, 1, 1, ustg_v)
        fire(utabt_hbm, uvn, 1, 1)
        drain(2)
        ext(ilane, 0, 2, istg_v)
        fire(itabt_hbm, ivn, 0, 2)
        drain(3)
        ext(ilane, 1, 3, istg_v)
        fire(itabt_hbm, ivn, 1, 3)

        h = [wb(P_B1 + j) for j in range(H)]
        for d in range(D):
            u_d = ustg_v[d]
            i_d = istg_v[d]
            e_d = u_d * i_d
            for j in range(H):
                h[j] = (h[j]
                        + e_d * wb(d * H + j)
                        + u_d * wb((D + d) * H + j)
                        + i_d * wb((2 * D + d) * H + j))
        logit = wb(P_B2)
        for j in range(H):
            logit = logit + jnp.maximum(h[j], 0.0) * wb(P_W2 + j)
        out_v[pl.ds(g * 16, 16)] = 1.0 / (1.0 + jnp.exp(-logit))
        return carry

    lax.fori_loop(0, NG, rnd, 0)

    for w in range(4):
        drain(w)
    pltpu.sync_copy(out_v, out_hbm.at[pl.ds(base, BW)])


def kernel(user_inputs, item_inputs, user_table, item_table, W1, b1, W2, b2):
    flat = jnp.concatenate([
        W1.reshape(-1),
        b1.reshape(-1),
        W2.reshape(-1),
        b2.reshape(-1),
        jnp.zeros((111,), jnp.float32),
    ])
    params = flat.reshape(4, 128)
    y = _hhgr_sc(user_inputs.astype(jnp.int32), item_inputs.astype(jnp.int32),
                 user_table.T, item_table.T, params)
    return y.reshape(B, 1)
